# baseline (speedup 1.0000x reference)
import functools

import jax
import jax.numpy as jnp
from jax.experimental import pallas as pl
from jax.experimental.pallas import tpu as pltpu

B, C_IN, HW, P = 8, 3, 512, 16
HP = HW // P
N = HP * HP
K, D = 8192, 256
CPP = C_IN * P * P
OFFSET = 32000
BOI = OFFSET + K
EOI = OFFSET + K + 1

KBLK = 1024
KT = K // KBLK
NV = KBLK // 128
MB = 256
NT = N // MB
HPB = MB // HP
BIG = float(jnp.finfo(jnp.float32).max)


NSEM = 8


def _im2col_copies(xv_ref, stage_s, b, sems):
    copies = []
    for c in range(C_IN):
        for p1 in range(P):
            j = c * P + p1
            copies.append(pltpu.make_async_copy(
                xv_ref.at[b, c, :, p1, :, :],
                stage_s.at[j],
                sems.at[j % NSEM],
            ))
    return copies


def _vq_kernel(xv_ref, wt_ref, cbt_ref, idx_ref, ef_ref,
               stage_a, stage_b, z_s, cn_s, vacc_s, iacc_s, sems):
    b = pl.program_id(0)
    kt = pl.program_id(1)

    @pl.when(b == 0)
    def _():
        cbt = cbt_ref[...]
        cn_s[:, pl.ds(kt * KBLK, KBLK)] = jnp.sum(cbt * cbt, axis=0,
                                                  keepdims=True)

    buf = jax.lax.rem(b, 2)

    @pl.when(jnp.logical_and(b == 0, kt == 0))
    def _():
        for cp in _im2col_copies(xv_ref, stage_a, 0, sems):
            cp.start()

    def _image_step(stage_cur, stage_next):
        @pl.when(kt == 0)
        def _():
            for cp in _im2col_copies(xv_ref, stage_cur, b, sems):
                cp.wait()

        nxt = _im2col_copies(xv_ref, stage_next, b + 1, sems)
        per = len(nxt) // KT
        for s in range(KT):
            @pl.when(jnp.logical_and(kt == s, b < B - 1))
            def _(s=s):
                for cp in nxt[s * per:(s + 1) * per]:
                    cp.start()

        @pl.when(kt == 0)
        def _():
            def zbody(i, carry):
                r8 = pl.ds(i * HPB, HPB)
                zi = jnp.zeros((MB, D), dtype=jnp.float32)
                for j in range(CPP // P):
                    lhs = stage_cur[j, r8, :, :].reshape(MB, P)
                    zi = zi + jnp.dot(lhs, wt_ref[pl.ds(j * P, P), :],
                                      preferred_element_type=jnp.float32)
                r = pl.ds(i * MB, MB)
                ef_ref[0, r, :] = zi
                z_s[r, :] = zi * (-2.0)
                return carry
            jax.lax.fori_loop(0, NT, zbody, 0)
            vacc_s[...] = jnp.full((N, 128), BIG, dtype=jnp.float32)
            iacc_s[...] = jnp.zeros((N, 128), dtype=jnp.int32)

    @pl.when(buf == 0)
    def _():
        _image_step(stage_a, stage_b)

    @pl.when(buf == 1)
    def _():
        _image_step(stage_b, stage_a)

    cbt = cbt_ref[...]
    cn = cn_s[:, pl.ds(kt * KBLK, KBLK)]
    lane = jax.lax.broadcasted_iota(jnp.int32, (MB, 128), 1)

    def body(i, carry):
        r = pl.ds(i * MB, MB)
        scores = jnp.dot(z_s[r, :], cbt,
                         preferred_element_type=jnp.float32)
        dist = scores + cn
        vacc = vacc_s[r, :]
        iacc = iacc_s[r, :]
        for v in range(NV):
            dv = dist[:, v * 128:(v + 1) * 128]
            iv = lane + (v * 128 + kt * KBLK)
            better = dv < vacc
            vacc = jnp.where(better, dv, vacc)
            iacc = jnp.where(better, iv, iacc)
        vacc_s[r, :] = vacc
        iacc_s[r, :] = iacc

        @pl.when(kt == KT - 1)
        def _():
            gmin = jnp.min(vacc, axis=1, keepdims=True)
            cand = jnp.where(vacc == gmin, iacc, K)
            idx_ref[0, r, :] = (jnp.min(cand, axis=1, keepdims=True)
                                + OFFSET)
        return carry
    jax.lax.fori_loop(0, NT, body, 0)


@functools.partial(jax.jit, static_argnames=())
def kernel(x, W_enc, codebook):
    xv = x.reshape(B, C_IN, HP, P, HP, P)
    wt = W_enc.T
    cbt = codebook.T

    idx, ef = pl.pallas_call(
        _vq_kernel,
        grid=(B, KT),
        in_specs=[
            pl.BlockSpec(memory_space=pltpu.MemorySpace.HBM),
            pl.BlockSpec((CPP, D), lambda b, kt: (0, 0)),
            pl.BlockSpec((D, KBLK), lambda b, kt: (0, kt)),
        ],
        out_specs=[
            pl.BlockSpec((1, N, 1), lambda b, kt: (b, 0, 0)),
            pl.BlockSpec((1, N, D), lambda b, kt: (b, 0, 0)),
        ],
        out_shape=[
            jax.ShapeDtypeStruct((B, N, 1), jnp.int32),
            jax.ShapeDtypeStruct((B, N, D), jnp.float32),
        ],
        scratch_shapes=[
            pltpu.VMEM((CPP // P, HP, HP, P), jnp.float32),
            pltpu.VMEM((CPP // P, HP, HP, P), jnp.float32),
            pltpu.VMEM((N, D), jnp.float32),
            pltpu.VMEM((1, K), jnp.float32),
            pltpu.VMEM((N, 128), jnp.float32),
            pltpu.VMEM((N, 128), jnp.int32),
            pltpu.SemaphoreType.DMA((NSEM,)),
        ],
        compiler_params=pltpu.CompilerParams(
            dimension_semantics=("arbitrary", "arbitrary"),
        ),
    )(xv, wt, cbt)

    indices = idx.reshape(B, N)[None]
    boi = jnp.full((1, B, 1), BOI, dtype=jnp.int32)
    eoi = jnp.full((1, B, 1), EOI, dtype=jnp.int32)
    input_ids = jnp.concatenate([boi, indices, eoi], axis=-1)
    attention_mask = jnp.ones((B, N + 2), dtype=jnp.int32)
    return input_ids, attention_mask, ef


# speedup vs baseline: 1.1010x; 1.1010x over previous
import functools

import jax
import jax.numpy as jnp
from jax.experimental import pallas as pl
from jax.experimental.pallas import tpu as pltpu

B, C_IN, HW, P = 8, 3, 512, 16
HP = HW // P
N = HP * HP
K, D = 8192, 256
CPP = C_IN * P * P
OFFSET = 32000
BOI = OFFSET + K
EOI = OFFSET + K + 1

KBLK = 1024
KT = K // KBLK
NV = KBLK // 128
MB = 512
NT = N // MB
HPB = MB // HP
BIG = float(jnp.finfo(jnp.float32).max)


NSEM = 8


def _im2col_copies(xv_ref, stage_s, b, sems):
    copies = []
    for c in range(C_IN):
        for p1 in range(P):
            j = c * P + p1
            copies.append(pltpu.make_async_copy(
                xv_ref.at[b, c, :, p1, :, :],
                stage_s.at[j],
                sems.at[j % NSEM],
            ))
    return copies


def _vq_kernel(xv_ref, wt_ref, cbt_ref, idx_ref, ef_ref,
               stage_a, stage_b, z_s, cn_s, vacc_s, iacc_s, sems):
    b = pl.program_id(0)
    kt = pl.program_id(1)

    @pl.when(b == 0)
    def _():
        cbt = cbt_ref[...]
        cn_s[:, pl.ds(kt * KBLK, KBLK)] = jnp.sum(cbt * cbt, axis=0,
                                                  keepdims=True)

    buf = jax.lax.rem(b, 2)

    @pl.when(jnp.logical_and(b == 0, kt == 0))
    def _():
        for cp in _im2col_copies(xv_ref, stage_a, 0, sems):
            cp.start()

    def _image_step(stage_cur, stage_next):
        @pl.when(kt == 0)
        def _():
            for cp in _im2col_copies(xv_ref, stage_cur, b, sems):
                cp.wait()

        @pl.when(jnp.logical_and(kt == 0, b < B - 1))
        def _():
            for cp in _im2col_copies(xv_ref, stage_next, b + 1, sems):
                cp.start()

        @pl.when(kt == 0)
        def _():
            def zbody(i, carry):
                r8 = pl.ds(i * HPB, HPB)
                zi = jnp.zeros((MB, D), dtype=jnp.float32)
                for j in range(CPP // P):
                    lhs = stage_cur[j, r8, :, :].reshape(MB, P)
                    zi = zi + jnp.dot(lhs, wt_ref[pl.ds(j * P, P), :],
                                      preferred_element_type=jnp.float32)
                r = pl.ds(i * MB, MB)
                ef_ref[0, r, :] = zi
                z_s[r, :] = zi * (-2.0)
                return carry
            jax.lax.fori_loop(0, NT, zbody, 0)
            vacc_s[...] = jnp.full((N, 128), BIG, dtype=jnp.float32)
            iacc_s[...] = jnp.zeros((N, 128), dtype=jnp.int32)

    @pl.when(buf == 0)
    def _():
        _image_step(stage_a, stage_b)

    @pl.when(buf == 1)
    def _():
        _image_step(stage_b, stage_a)

    cbt = cbt_ref[...]
    cn = cn_s[:, pl.ds(kt * KBLK, KBLK)]
    lane = jax.lax.broadcasted_iota(jnp.int32, (MB, 128), 1)

    def body(i, carry):
        r = pl.ds(i * MB, MB)
        scores = jnp.dot(z_s[r, :], cbt,
                         preferred_element_type=jnp.float32)
        dist = scores + cn
        vacc = vacc_s[r, :]
        iacc = iacc_s[r, :]
        for v in range(NV):
            dv = dist[:, v * 128:(v + 1) * 128]
            iv = lane + (v * 128 + kt * KBLK)
            better = dv < vacc
            vacc = jnp.where(better, dv, vacc)
            iacc = jnp.where(better, iv, iacc)
        vacc_s[r, :] = vacc
        iacc_s[r, :] = iacc

        @pl.when(kt == KT - 1)
        def _():
            gmin = jnp.min(vacc, axis=1, keepdims=True)
            cand = jnp.where(vacc == gmin, iacc, K)
            idx_ref[0, r, :] = (jnp.min(cand, axis=1, keepdims=True)
                                + OFFSET)
        return carry
    jax.lax.fori_loop(0, NT, body, 0)


@functools.partial(jax.jit, static_argnames=())
def kernel(x, W_enc, codebook):
    xv = x.reshape(B, C_IN, HP, P, HP, P)
    wt = W_enc.T
    cbt = codebook.T

    idx, ef = pl.pallas_call(
        _vq_kernel,
        grid=(B, KT),
        in_specs=[
            pl.BlockSpec(memory_space=pltpu.MemorySpace.HBM),
            pl.BlockSpec((CPP, D), lambda b, kt: (0, 0)),
            pl.BlockSpec((D, KBLK), lambda b, kt: (0, kt)),
        ],
        out_specs=[
            pl.BlockSpec((1, N, 1), lambda b, kt: (b, 0, 0)),
            pl.BlockSpec((1, N, D), lambda b, kt: (b, 0, 0)),
        ],
        out_shape=[
            jax.ShapeDtypeStruct((B, N, 1), jnp.int32),
            jax.ShapeDtypeStruct((B, N, D), jnp.float32),
        ],
        scratch_shapes=[
            pltpu.VMEM((CPP // P, HP, HP, P), jnp.float32),
            pltpu.VMEM((CPP // P, HP, HP, P), jnp.float32),
            pltpu.VMEM((N, D), jnp.float32),
            pltpu.VMEM((1, K), jnp.float32),
            pltpu.VMEM((N, 128), jnp.float32),
            pltpu.VMEM((N, 128), jnp.int32),
            pltpu.SemaphoreType.DMA((NSEM,)),
        ],
        compiler_params=pltpu.CompilerParams(
            dimension_semantics=("arbitrary", "arbitrary"),
        ),
    )(xv, wt, cbt)

    indices = idx.reshape(B, N)[None]
    boi = jnp.full((1, B, 1), BOI, dtype=jnp.int32)
    eoi = jnp.full((1, B, 1), EOI, dtype=jnp.int32)
    input_ids = jnp.concatenate([boi, indices, eoi], axis=-1)
    attention_mask = jnp.ones((B, N + 2), dtype=jnp.int32)
    return input_ids, attention_mask, ef
